# baseline (speedup 1.0000x reference)
import functools

import jax
import jax.numpy as jnp
from jax import lax
from jax.experimental import pallas as pl
from jax.experimental.pallas import tpu as pltpu
from jax.experimental.pallas import tpu_sc as plsc

NUM_SETS = 1000000
EMBED_DIM = 64
HIDDEN_DIM = 128
BATCH = 16384
TEAM = 6

NC, NS = 2, 16
NW = NC * NS
TEAMS_PER_W = BATCH // NW
TEAMS_PER_CHUNK = 16
CHUNKS = TEAMS_PER_W // TEAMS_PER_CHUNK
IDX_PER_CHUNK = TEAMS_PER_CHUNK * TEAM
LANES = 16
TROWS = 512
PHYS_ROWS = 977 * TROWS


def _tc_relayout(table_t):
    grid = PHYS_ROWS // TROWS
    koff = PHYS_ROWS // TROWS

    def body(x1_ref, x2_ref, o_ref):
        o_ref[:, 0:EMBED_DIM] = x1_ref[...].T
        o_ref[:, EMBED_DIM : 2 * EMBED_DIM] = x2_ref[...].T

    return pl.pallas_call(
        body,
        grid=(grid,),
        in_specs=[
            pl.BlockSpec((EMBED_DIM, TROWS), lambda i: (0, i)),
            pl.BlockSpec((EMBED_DIM, TROWS), lambda i: (0, i + koff)),
        ],
        out_specs=pl.BlockSpec((TROWS, 2 * EMBED_DIM), lambda i: (i, 0)),
        out_shape=jax.ShapeDtypeStruct((PHYS_ROWS, 2 * EMBED_DIM), jnp.float32),
    )(table_t, table_t)


def _sc_pool(pidx3, h643, table2):
    mesh = plsc.VectorSubcoreMesh(core_axis_name="c", subcore_axis_name="s")

    @functools.partial(
        pl.kernel,
        out_type=jax.ShapeDtypeStruct((BATCH, EMBED_DIM), jnp.float32),
        mesh=mesh,
        scratch_types=[
            pltpu.VMEM((CHUNKS, IDX_PER_CHUNK), jnp.int32),
            pltpu.VMEM((CHUNKS, TEAM, TEAMS_PER_CHUNK), jnp.int32),
            pltpu.VMEM((IDX_PER_CHUNK, 2 * EMBED_DIM), jnp.float32),
            pltpu.VMEM((TEAMS_PER_W, EMBED_DIM), jnp.float32),
            pltpu.SemaphoreType.DMA,
        ],
        compiler_params=pltpu.CompilerParams(use_tc_tiling_on_sc=False),
    )
    def k(pidx_hbm, h64_hbm, table_hbm, out_hbm, pidx_v, h_v, rows_v, out_v, sem):
        wid = lax.axis_index("s") * NC + lax.axis_index("c")
        pltpu.sync_copy(pidx_hbm.at[wid], pidx_v)
        pltpu.sync_copy(h64_hbm.at[wid], h_v)

        def chunk_body(j, carry):
            pltpu.async_copy(table_hbm.at[pidx_v.at[j]], rows_v, sem).wait()
            hvecs = [h_v[j, r] for r in range(TEAM)]
            for t in range(TEAMS_PER_CHUNK):
                row0 = t * TEAM
                hs = [hvecs[r][t] for r in range(TEAM)]
                for cb in range(EMBED_DIM // LANES):
                    acc = rows_v[row0, pl.ds(hs[0] + cb * LANES, LANES)]
                    for r in range(1, TEAM):
                        acc = acc + rows_v[row0 + r, pl.ds(hs[r] + cb * LANES, LANES)]
                    out_v[
                        j * TEAMS_PER_CHUNK + t, pl.ds(cb * LANES, LANES)
                    ] = acc * (1.0 / TEAM)
            return carry

        lax.fori_loop(0, CHUNKS, chunk_body, 0)
        pltpu.sync_copy(out_v, out_hbm.at[pl.ds(wid * TEAMS_PER_W, TEAMS_PER_W)])

    return k(pidx3, h643, table2)


def _tc_mlp(x, w1t, b1, w2t, b2):
    bb = 1024

    def body(x_ref, w1_ref, b1_ref, w2_ref, b2_ref, o_ref):
        h = jnp.dot(x_ref[...], w1_ref[...], preferred_element_type=jnp.float32)
        h = jnp.maximum(h + b1_ref[...], 0.0)
        o_ref[...] = (
            jnp.dot(h, w2_ref[...], preferred_element_type=jnp.float32) + b2_ref[...]
        )

    return pl.pallas_call(
        body,
        grid=(BATCH // bb,),
        in_specs=[
            pl.BlockSpec((bb, EMBED_DIM), lambda i: (i, 0)),
            pl.BlockSpec((EMBED_DIM, HIDDEN_DIM), lambda i: (0, 0)),
            pl.BlockSpec((1, HIDDEN_DIM), lambda i: (0, 0)),
            pl.BlockSpec((HIDDEN_DIM, 1), lambda i: (0, 0)),
            pl.BlockSpec((1, 1), lambda i: (0, 0)),
        ],
        out_specs=pl.BlockSpec((bb, 1), lambda i: (i, 0)),
        out_shape=jax.ShapeDtypeStruct((BATCH, 1), jnp.float32),
    )(x, w1t, b1, w2t, b2)


def kernel(team_indices, embedding, fc1_w, fc1_b, fc2_w, fc2_b):
    idx = team_indices.astype(jnp.int32)
    hi = (idx >= PHYS_ROWS).astype(jnp.int32)
    pidx3 = (idx - hi * PHYS_ROWS).reshape(NW, CHUNKS, IDX_PER_CHUNK)
    h643 = jnp.transpose(
        (hi * EMBED_DIM).reshape(NW, CHUNKS, TEAMS_PER_CHUNK, TEAM),
        (0, 1, 3, 2),
    )
    table2 = _tc_relayout(embedding.T)
    pooled = _sc_pool(pidx3, h643, table2)
    out = _tc_mlp(
        pooled,
        fc1_w.T,
        fc1_b.reshape(1, HIDDEN_DIM),
        fc2_w.T,
        fc2_b.reshape(1, 1),
    )
    return out[:, 0]


# speedup vs baseline: 2.5623x; 2.5623x over previous
import functools

import jax
import jax.numpy as jnp
from jax import lax
from jax.experimental import pallas as pl
from jax.experimental.pallas import tpu as pltpu
from jax.experimental.pallas import tpu_sc as plsc

NUM_SETS = 1000000
EMBED_DIM = 64
HIDDEN_DIM = 128
BATCH = 16384
TEAM = 6

NC, NS = 2, 16
NW = NC * NS
TEAMS_PER_W = BATCH // NW
TEAMS_PER_CHUNK = 16
CHUNKS = TEAMS_PER_W // TEAMS_PER_CHUNK
IDX_PER_CHUNK = TEAMS_PER_CHUNK * TEAM
LANES = 16
NPAIR = 3908
PHYS_ROWS = NPAIR * 128


def _sc_relayout(table_t):
    mesh = plsc.VectorSubcoreMesh(core_axis_name="c", subcore_axis_name="s")

    @functools.partial(
        pl.kernel,
        out_type=jax.ShapeDtypeStruct((PHYS_ROWS, 2 * EMBED_DIM), jnp.float32),
        mesh=mesh,
        scratch_types=[
            pltpu.VMEM((2, 2, EMBED_DIM, 128), jnp.float32),
            pltpu.VMEM((2, 128, 2 * EMBED_DIM), jnp.float32),
            pltpu.SemaphoreType.DMA((2,)),
            pltpu.SemaphoreType.DMA((2,)),
        ],
        compiler_params=pltpu.CompilerParams(
            use_tc_tiling_on_sc=True, needs_layout_passes=False
        ),
    )
    def k(t_hbm, out_hbm, in_v, out_v, isem, osem):
        wid = lax.axis_index("s") * NC + lax.axis_index("c")
        iota = lax.iota(jnp.int32, LANES)
        niter2 = ((NPAIR + NW - 1) // NW + 1) // 2

        def fire_in(i, s):
            pc = jnp.minimum(wid + i * NW, NPAIR - 1)
            pltpu.async_copy(
                t_hbm.at[:, pl.ds(pc * 128, 128)], in_v.at[s, 0], isem.at[s]
            )
            cshi = jnp.minimum(pc, NPAIR - 4) * 128 + PHYS_ROWS
            pltpu.async_copy(
                t_hbm.at[:, pl.ds(cshi, 128)], in_v.at[s, 1], isem.at[s]
            )

        fire_in(0, 0)
        fire_in(1, 1)

        def body(i2, carry):
            for s in range(2):
                i = i2 * 2 + s
                pw = jnp.minimum(wid + i * NW, NPAIR - 1)
                for h in range(2):
                    pltpu.make_async_copy(
                        t_hbm.at[:, pl.ds(0, 128)], in_v.at[s, h], isem.at[s]
                    ).wait()

                @pl.when(i2 >= 1)
                def _wait_out():
                    pltpu.make_async_copy(
                        out_v.at[s], out_hbm.at[pl.ds(0, 128), :], osem.at[s]
                    ).wait()

                @plsc.parallel_loop(0, LANES, unroll=2)
                def _kk_body(kk):
                    rot = (iota + kk) & (LANES - 1)
                    for half in range(2):
                        for cb in range(EMBED_DIM // LANES):
                            c_vec = cb * LANES + iota
                            oc_vec = half * EMBED_DIM + c_vec
                            xvs = [xb * LANES + rot for xb in range(128 // LANES)]
                            vs = [
                                plsc.load_gather(in_v.at[s, half], [c_vec, xv])
                                for xv in xvs
                            ]
                            for xv, v in zip(xvs, vs):
                                plsc.store_scatter(out_v.at[s], [xv, oc_vec], v)
                pltpu.async_copy(
                    out_v.at[s], out_hbm.at[pl.ds(pw * 128, 128), :], osem.at[s]
                )

                @pl.when(i + 2 <= 2 * niter2 - 1)
                def _refire():
                    fire_in(i + 2, s)

            return carry

        lax.fori_loop(0, niter2, body, 0)
        for s in range(2):
            pltpu.make_async_copy(
                out_v.at[s], out_hbm.at[pl.ds(0, 128), :], osem.at[s]
            ).wait()

    return k(table_t)


def _sc_pool(pidx3, h643, table2):
    mesh = plsc.VectorSubcoreMesh(core_axis_name="c", subcore_axis_name="s")

    @functools.partial(
        pl.kernel,
        out_type=jax.ShapeDtypeStruct((BATCH, EMBED_DIM), jnp.float32),
        mesh=mesh,
        scratch_types=[
            pltpu.VMEM((CHUNKS, IDX_PER_CHUNK), jnp.int32),
            pltpu.VMEM((CHUNKS, TEAM, TEAMS_PER_CHUNK), jnp.int32),
            pltpu.VMEM((2, IDX_PER_CHUNK, 2 * EMBED_DIM), jnp.float32),
            pltpu.VMEM((TEAMS_PER_W, EMBED_DIM), jnp.float32),
            pltpu.SemaphoreType.DMA((2,)),
        ],
        compiler_params=pltpu.CompilerParams(use_tc_tiling_on_sc=False),
    )
    def k(pidx_hbm, h64_hbm, table_hbm, out_hbm, pidx_v, h_v, rows_v, out_v, sem):
        wid = lax.axis_index("s") * NC + lax.axis_index("c")
        pltpu.sync_copy(pidx_hbm.at[wid], pidx_v)
        pltpu.sync_copy(h64_hbm.at[wid], h_v)

        def fire(j, s):
            jc = jnp.minimum(j, CHUNKS - 1)
            pltpu.async_copy(table_hbm.at[pidx_v.at[jc]], rows_v.at[s], sem.at[s])

        fire(0, 0)
        fire(1, 1)

        def chunk_body(j2, carry):
            for s in range(2):
                j = j2 * 2 + s
                pltpu.make_async_copy(
                    table_hbm.at[pidx_v.at[0]], rows_v.at[s], sem.at[s]
                ).wait()
                hvecs = [h_v[j, r] for r in range(TEAM)]
                for t in range(TEAMS_PER_CHUNK):
                    row0 = t * TEAM
                    hs = [hvecs[r][t] for r in range(TEAM)]
                    for cb in range(EMBED_DIM // LANES):
                        acc = rows_v[s, row0, pl.ds(hs[0] + cb * LANES, LANES)]
                        for r in range(1, TEAM):
                            acc = acc + rows_v[
                                s, row0 + r, pl.ds(hs[r] + cb * LANES, LANES)
                            ]
                        out_v[
                            j * TEAMS_PER_CHUNK + t, pl.ds(cb * LANES, LANES)
                        ] = acc * (1.0 / TEAM)

                @pl.when(j + 2 <= CHUNKS - 1)
                def _refire():
                    fire(j + 2, s)

            return carry

        lax.fori_loop(0, CHUNKS // 2, chunk_body, 0)
        pltpu.sync_copy(out_v, out_hbm.at[pl.ds(wid * TEAMS_PER_W, TEAMS_PER_W)])

    return k(pidx3, h643, table2)


def _tc_mlp(x, w1t, b1, w2t, b2):
    bb = 1024

    def body(x_ref, w1_ref, b1_ref, w2_ref, b2_ref, o_ref):
        h = jnp.dot(x_ref[...], w1_ref[...], preferred_element_type=jnp.float32)
        h = jnp.maximum(h + b1_ref[...], 0.0)
        o_ref[...] = (
            jnp.dot(h, w2_ref[...], preferred_element_type=jnp.float32) + b2_ref[...]
        )

    return pl.pallas_call(
        body,
        grid=(BATCH // bb,),
        in_specs=[
            pl.BlockSpec((bb, EMBED_DIM), lambda i: (i, 0)),
            pl.BlockSpec((EMBED_DIM, HIDDEN_DIM), lambda i: (0, 0)),
            pl.BlockSpec((1, HIDDEN_DIM), lambda i: (0, 0)),
            pl.BlockSpec((HIDDEN_DIM, 1), lambda i: (0, 0)),
            pl.BlockSpec((1, 1), lambda i: (0, 0)),
        ],
        out_specs=pl.BlockSpec((bb, 1), lambda i: (i, 0)),
        out_shape=jax.ShapeDtypeStruct((BATCH, 1), jnp.float32),
    )(x, w1t, b1, w2t, b2)


def kernel(team_indices, embedding, fc1_w, fc1_b, fc2_w, fc2_b):
    idx = team_indices.astype(jnp.int32)
    hi = (idx >= PHYS_ROWS).astype(jnp.int32)
    pidx3 = (idx - hi * PHYS_ROWS).reshape(NW, CHUNKS, IDX_PER_CHUNK)
    h643 = jnp.transpose(
        (hi * EMBED_DIM).reshape(NW, CHUNKS, TEAMS_PER_CHUNK, TEAM),
        (0, 1, 3, 2),
    )
    table2 = _sc_relayout(embedding.T)
    pooled = _sc_pool(pidx3, h643, table2)
    out = _tc_mlp(
        pooled,
        fc1_w.T,
        fc1_b.reshape(1, HIDDEN_DIM),
        fc2_w.T,
        fc2_b.reshape(1, 1),
    )
    return out[:, 0]
